# baseline (speedup 1.0000x reference)
import functools

import jax
import jax.numpy as jnp
from jax import lax
from jax.experimental import pallas as pl
from jax.experimental.pallas import tpu as pltpu
from jax.experimental.pallas import tpu_sc as plsc

N_NODES = 10000
N_EDGES = 320000
D = 128
D_EDGE = 16
STEPS = 3

F32 = jnp.float32


_NODE_BLK = 1000
_EDGE_BLK = 1600


def _dot(a, b):
    return jnp.dot(a, b, preferred_element_type=F32)


_dot_a = _dot


def _enc_body(x_ref, w0, b0, w1, b1, w2, b2, h_ref):
    a = jnp.maximum(_dot(x_ref[...], w0[...]) + b0[...], 0.0)
    a = jnp.maximum(_dot_a(a, w1[...]) + b1[...], 0.0)
    h_ref[...] = _dot_a(a, w2[...]) + b2[...]


def _enc(x, w0, b0, w1, b1, w2, b2):
    g = N_NODES // _NODE_BLK
    row = pl.BlockSpec((_NODE_BLK, D), lambda i: (i, 0))
    wspec = pl.BlockSpec((D, D), lambda i: (0, 0))
    bspec = pl.BlockSpec((1, D), lambda i: (0, 0))
    return pl.pallas_call(
        _enc_body,
        grid=(g,),
        in_specs=[row, wspec, bspec, wspec, bspec, wspec, bspec],
        out_specs=row,
        out_shape=jax.ShapeDtypeStruct((N_NODES, D), F32),
    )(x, w0, b0, w1, b1, w2, b2)


def _edge_body(ea_ref, w0, b0, w1, b1, w2, b2, e_ref):
    a = jnp.maximum(_dot(ea_ref[...], w0[...]) + b0[...], 0.0)
    a = jnp.maximum(_dot_a(a, w1[...]) + b1[...], 0.0)
    e_ref[...] = _dot_a(a, w2[...]) + b2[...]


def _edge(ea, w0, b0, w1, b1, w2, b2):
    g = N_EDGES // _EDGE_BLK
    row = pl.BlockSpec((_EDGE_BLK, D), lambda i: (i, 0))
    return pl.pallas_call(
        _edge_body,
        grid=(g,),
        in_specs=[
            pl.BlockSpec((_EDGE_BLK, D_EDGE), lambda i: (i, 0)),
            pl.BlockSpec((D_EDGE, D), lambda i: (0, 0)),
            pl.BlockSpec((1, D), lambda i: (0, 0)),
            pl.BlockSpec((D, D), lambda i: (0, 0)),
            pl.BlockSpec((1, D), lambda i: (0, 0)),
            pl.BlockSpec((D, D), lambda i: (0, 0)),
            pl.BlockSpec((1, D), lambda i: (0, 0)),
        ],
        out_specs=row,
        out_shape=jax.ShapeDtypeStruct((N_EDGES, D), F32),
    )(ea, w0, b0, w1, b1, w2, b2)


def _msg_body(xi_ref, xj_ref, e_ref, w1, b1, w2, b2, w3, b3, m_ref):
    xi = xi_ref[...]
    d = xj_ref[...] - xi
    w1f = w1[...]
    z = jnp.maximum(_dot_a(xi, w1f[0:D, :]) + _dot_a(d, w1f[D:2 * D, :])
                    + _dot_a(e_ref[...], w1f[2 * D:3 * D, :]) + b1[...], 0.0)
    z = jnp.maximum(_dot_a(z, w2[...]) + b2[...], 0.0)
    m_ref[...] = _dot_a(z, w3[...]) + b3[...]


def _msg(xi, xj, e, w1, b1, w2, b2, w3, b3):
    g = N_EDGES // _EDGE_BLK
    row = pl.BlockSpec((_EDGE_BLK, D), lambda i: (i, 0))
    w1spec = pl.BlockSpec((3 * D, D), lambda i: (0, 0))
    wspec = pl.BlockSpec((D, D), lambda i: (0, 0))
    bspec = pl.BlockSpec((1, D), lambda i: (0, 0))
    return pl.pallas_call(
        _msg_body,
        grid=(g,),
        in_specs=[row, row, row, w1spec, bspec, wspec, bspec, wspec, bspec],
        out_specs=row,
        out_shape=jax.ShapeDtypeStruct((N_EDGES, D), F32),
    )(xi, xj, e, w1, b1, w2, b2, w3, b3)


def _node_body(a0_ref, a1_ref, h_ref, w0, b0, w1, b1, w2, b2, hn_ref):
    agg = a0_ref[0] + a1_ref[0]
    h = h_ref[...]
    w0f = w0[...]
    z = jnp.maximum(_dot_a(agg, w0f[0:D, :]) + _dot_a(h, w0f[D:2 * D, :]) + b0[...], 0.0)
    z = jnp.maximum(_dot_a(z, w1[...]) + b1[...], 0.0)
    hn_ref[...] = _dot_a(z, w2[...]) + b2[...] + h


def _node(aggs, h, w0, b0, w1, b1, w2, b2):
    g = N_NODES // _NODE_BLK
    row = pl.BlockSpec((_NODE_BLK, D), lambda i: (i, 0))
    agg0 = pl.BlockSpec((1, _NODE_BLK, D), lambda i: (0, i, 0))
    agg1 = pl.BlockSpec((1, _NODE_BLK, D), lambda i: (1, i, 0))
    wspec = pl.BlockSpec((D, D), lambda i: (0, 0))
    bspec = pl.BlockSpec((1, D), lambda i: (0, 0))
    w0spec = pl.BlockSpec((2 * D, D), lambda i: (0, 0))
    return pl.pallas_call(
        _node_body,
        grid=(g,),
        in_specs=[agg0, agg1, row, w0spec, bspec, wspec, bspec, wspec, bspec],
        out_specs=row,
        out_shape=jax.ShapeDtypeStruct((N_NODES, D), F32),
    )(aggs, aggs, h, w0, b0, w1, b1, w2, b2)


def _dec_body(h_ref, w0, b0, w1, b1, w2, b2, o_ref):
    a = jnp.maximum(_dot_a(h_ref[...], w0[...]) + b0[...], 0.0)
    a = jnp.maximum(_dot_a(a, w1[...]) + b1[...], 0.0)
    o_ref[...] = _dot_a(a, w2[...]) + b2[...]


def _dec(h, w0, b0, w1, b1, w2, b2):
    g = N_NODES // _NODE_BLK
    row = pl.BlockSpec((_NODE_BLK, D), lambda i: (i, 0))
    wspec = pl.BlockSpec((D, D), lambda i: (0, 0))
    bspec = pl.BlockSpec((1, D), lambda i: (0, 0))
    return pl.pallas_call(
        _dec_body,
        grid=(g,),
        in_specs=[row, wspec, bspec, wspec, bspec, wspec, bspec],
        out_specs=row,
        out_shape=jax.ShapeDtypeStruct((N_NODES, D), F32),
    )(h, w0, b0, w1, b1, w2, b2)


_NC = 2
_NS = 16
_NW = _NC * _NS
_EPW = N_EDGES // _NW
_C = 80
_NCHUNK = _EPW // _C
_NPAD = 10240
_RPS = _NPAD // _NS

@functools.cache
def _sc_kernels():
    mesh = plsc.VectorSubcoreMesh(core_axis_name="c", subcore_axis_name="s",
                                  num_cores=_NC, num_subcores=_NS)

    @functools.partial(
        pl.kernel,
        out_type=(
            jax.ShapeDtypeStruct((N_EDGES, D), F32),
            jax.ShapeDtypeStruct((N_EDGES, D), F32),
        ),
        mesh=mesh,
        scratch_types=[
            pltpu.VMEM((_C,), jnp.int32),
            pltpu.VMEM((_C,), jnp.int32),
            pltpu.VMEM((_C, D), F32),
            pltpu.VMEM((_C, D), F32),
            pltpu.SemaphoreType.DMA,
            pltpu.SemaphoreType.DMA,
        ],
    )
    def sc_gather(p_hbm, q_hbm, src_hbm, dst_hbm, gp_hbm, gq_hbm,
                  di, si, pr, qr, sem1, sem2):
        wid = lax.axis_index("s") * _NC + lax.axis_index("c")
        base = wid * _EPW

        def chunk(i, carry):
            off = base + i * _C
            pltpu.sync_copy(dst_hbm.at[pl.ds(off, _C)], di)
            pltpu.sync_copy(src_hbm.at[pl.ds(off, _C)], si)
            cp1 = pltpu.async_copy(p_hbm.at[di], pr, sem1)
            cp2 = pltpu.async_copy(q_hbm.at[si], qr, sem2)
            cp1.wait()
            cp2.wait()
            pltpu.sync_copy(pr, gp_hbm.at[pl.ds(off, _C)])
            pltpu.sync_copy(qr, gq_hbm.at[pl.ds(off, _C)])
            return carry

        lax.fori_loop(0, _NCHUNK, chunk, 0)

    @functools.partial(
        pl.kernel,
        out_type=jax.ShapeDtypeStruct((2 * _NPAD, D), F32),
        mesh=mesh,
        scratch_types=[
            pltpu.VMEM((_C,), jnp.int32),
            pltpu.VMEM((_C, D), F32),
            pltpu.VMEM_SHARED((_NPAD, D), F32),
        ],
    )
    def sc_scatter(m_hbm, dst_hbm, zeros_hbm, out_hbm, di, mv, acc):
        cid = lax.axis_index("c")
        sid = lax.axis_index("s")
        wid = sid * _NC + cid
        pltpu.sync_copy(zeros_hbm.at[pl.ds(sid * _RPS, _RPS)],
                        acc.at[pl.ds(sid * _RPS, _RPS)])
        plsc.subcore_barrier()

        base = wid * _EPW

        def chunk(i, carry):
            off = base + i * _C
            pltpu.sync_copy(dst_hbm.at[pl.ds(off, _C)], di)
            pltpu.sync_copy(m_hbm.at[pl.ds(off, _C)], mv)
            pltpu.sync_copy(mv, acc.at[di], add=True)
            return carry

        lax.fori_loop(0, _NCHUNK, chunk, 0)
        plsc.subcore_barrier()
        pltpu.sync_copy(acc.at[pl.ds(sid * _RPS, _RPS)],
                        out_hbm.at[pl.ds(cid * _NPAD + sid * _RPS, _RPS)])

    return sc_gather, sc_scatter


def kernel(x, edge_index, edge_attr, params):
    src = edge_index[0]
    dst = edge_index[1]
    enc_w, enc_b = params["enc"]["w"], params["enc"]["b"]
    edge_w, edge_b = params["edge"]["w"], params["edge"]["b"]
    msg_w, msg_b = params["msg"]["w"], params["msg"]["b"]
    node_w, node_b = params["node"]["w"], params["node"]["b"]
    dec_w, dec_b = params["dec"]["w"], params["dec"]["b"]

    def b2(v):
        return v.reshape(1, D)

    h = _enc(x, enc_w[0], b2(enc_b[0]), enc_w[1], b2(enc_b[1]),
             enc_w[2], b2(enc_b[2]))
    e = _edge(edge_attr, edge_w[0], b2(edge_b[0]), edge_w[1], b2(edge_b[1]),
              edge_w[2], b2(edge_b[2]))
    zeros = jnp.zeros((_NPAD, D), F32)
    sc_gather, sc_scatter = _sc_kernels()
    for _ in range(STEPS):
        xi, xj = sc_gather(h, h, src, dst)
        m = _msg(xi, xj, e, msg_w[0], b2(msg_b[0]), msg_w[1], b2(msg_b[1]),
                 msg_w[2], b2(msg_b[2]))
        aggs = sc_scatter(m, dst, zeros).reshape(2, _NPAD, D)
        h = _node(aggs, h, node_w[0], b2(node_b[0]), node_w[1],
                  b2(node_b[1]), node_w[2], b2(node_b[2]))
    return _dec(h, dec_w[0], b2(dec_b[0]), dec_w[1], b2(dec_b[1]),
                dec_w[2], b2(dec_b[2]))
